# Initial kernel scaffold; baseline (speedup 1.0000x reference)
import jax
import jax.numpy as jnp
from jax.experimental import pallas as pl


def kernel(x, Wt, bt, gt, bet, Ws, bs, gs, bes, W2, b2):
    raise NotImplementedError("write your pallas kernel here")


# speedup vs baseline: 3.0626x; 3.0626x over previous
import functools

import jax
import jax.numpy as jnp
from jax import lax
from jax.experimental import pallas as pl
from jax.experimental.pallas import tpu as pltpu
from jax.experimental.pallas import tpu_sc as plsc

_B, _L, _D = 4, 2048, 1024
_E, _K, _KWIN = 16, 2, 25
_PAD = (_KWIN - 1) // 2
_NT = _B * _L
_NSUB = 16
_TPT = _NT // _NSUB
_CAP = 8.0
_ZR = 128


def _mavg_body(x_ref, o_ref):
    xb = x_ref[0]
    f = xb.shape[1]
    xp = jnp.concatenate(
        [jnp.broadcast_to(xb[0:1], (_PAD, f)), xb,
         jnp.broadcast_to(xb[_L - 1:_L], (_PAD, f))], axis=0)
    acc = xp[0:_L]
    for d in range(1, _KWIN):
        acc = acc + xp[d:d + _L]
    o_ref[0] = acc * (1.0 / _KWIN)


def _moving_mean(x, interpret=False):
    f = 512
    return pl.pallas_call(
        _mavg_body,
        grid=(_B, _D // f),
        in_specs=[pl.BlockSpec((1, _L, f), lambda b, i: (b, 0, i))],
        out_specs=pl.BlockSpec((1, _L, f), lambda b, i: (b, 0, i)),
        out_shape=jax.ShapeDtypeStruct((_B, _L, _D), jnp.float32),
        interpret=interpret,
    )(x)


_T = 512


def _dense_body(x_ref, mm_ref, Wt_ref, bt_ref, gt_ref, bet_ref,
                Ws_ref, bs_ref, gs_ref, bes_ref, W2_ref, b2_ref, med_ref):
    mmc = mm_ref[...]
    res = x_ref[...] - mmc
    h1 = jnp.dot(res, Wt_ref[...], preferred_element_type=jnp.float32) + bt_ref[...]
    mu1 = jnp.mean(h1, axis=1, keepdims=True)
    d1 = h1 - mu1
    v1 = jnp.mean(d1 * d1, axis=1, keepdims=True)
    ti = d1 * lax.rsqrt(v1 + 1e-5) * gt_ref[...] + bet_ref[...]
    h2 = jnp.dot(mmc, Ws_ref[...], preferred_element_type=jnp.float32) + bs_ref[...]
    mu2 = jnp.mean(h2, axis=1, keepdims=True)
    d2 = h2 - mu2
    v2 = jnp.mean(d2 * d2, axis=1, keepdims=True)
    si = d2 * lax.rsqrt(v2 + 1e-5) * gs_ref[...] + bes_ref[...]
    a = jnp.maximum(ti + si, 0.0)
    med_ref[...] = jnp.dot(a, W2_ref[...], preferred_element_type=jnp.float32) + b2_ref[...]


def _dense(x2, mm2, Wt, bt, gt, bet, Ws, bs, gs, bes, W2, b2, interpret=False):
    full = lambda i: (0, 0)
    return pl.pallas_call(
        _dense_body,
        grid=(_NT // _T,),
        in_specs=[
            pl.BlockSpec((_T, _D), lambda i: (i, 0)),
            pl.BlockSpec((_T, _D), lambda i: (i, 0)),
            pl.BlockSpec((_D, _D), full),
            pl.BlockSpec((1, _D), full),
            pl.BlockSpec((1, _D), full),
            pl.BlockSpec((1, _D), full),
            pl.BlockSpec((_D, _D), full),
            pl.BlockSpec((1, _D), full),
            pl.BlockSpec((1, _D), full),
            pl.BlockSpec((1, _D), full),
            pl.BlockSpec((_D, _E), full),
            pl.BlockSpec((1, _E), full),
        ],
        out_specs=pl.BlockSpec((_T, _E), lambda i: (i, 0)),
        out_shape=jax.ShapeDtypeStruct((_NT, _E), jnp.float32),
        interpret=interpret,
    )(x2, mm2, Wt, bt, gt, bet, Ws, bs, gs, bes, W2, b2)


@functools.cache
def _build_route_kernel():
    sc_mesh = plsc.VectorSubcoreMesh(
        core_axis_name="c", subcore_axis_name="s",
        num_cores=1, num_subcores=_NSUB)
    return functools.partial(
        pl.kernel,
        out_type=jax.ShapeDtypeStruct((_NT, _E), jnp.float32),
        mesh=sc_mesh,
        compiler_params=pltpu.CompilerParams(needs_layout_passes=False),
        scratch_types=[
        pltpu.VMEM((_TPT, _E), jnp.float32),
        pltpu.VMEM((2, _E), jnp.float32),
        pltpu.VMEM((_NSUB, 2, _E), jnp.float32),
        pltpu.VMEM((_E, _E), jnp.float32),
        pltpu.VMEM((_E, _E), jnp.float32),
        pltpu.VMEM((_E, _E), jnp.float32),
        pltpu.VMEM((_ZR, _E), jnp.float32),
        pltpu.VMEM_SHARED((_NSUB, 2, _E), jnp.float32),
        ],
    )(_route_body)


def _route_body(med_hbm, out_hbm, med_v, selp_v, comb_v, rows_v, g_v,
                ob_v, z_v, shared):
    wid = lax.axis_index("s")
    base = wid * _TPT
    pltpu.sync_copy(med_hbm.at[pl.ds(base, _TPT)], med_v)

    iota = lax.iota(jnp.int32, _E)
    zero = jnp.zeros((_E,), jnp.float32)
    neg = jnp.full((_E,), -3.4e38, jnp.float32)

    def tok(i, carry):
        s1, s2 = carry
        v = med_v[i]
        m1 = jnp.max(v)
        i1 = plsc.all_reduce_ffs(v == m1)
        oh1 = iota == i1
        v2 = jnp.where(oh1, neg, v)
        m2 = jnp.max(v2)
        i2 = plsc.all_reduce_ffs(v2 == m2)
        oh2 = iota == i2
        return jnp.where(oh1, 1.0, s1), jnp.where(oh2, 1.0, s2)

    s1, s2 = lax.fori_loop(0, _TPT, tok, (zero, zero), unroll=4)
    selp_v[0] = s1
    selp_v[1] = s2
    pltpu.sync_copy(selp_v, shared.at[wid])

    def zb(i, _):
        z_v[i] = zero
        return 0
    lax.fori_loop(0, _ZR, zb, 0, unroll=8)
    for k in range(_TPT // _ZR):
        pltpu.sync_copy(z_v, out_hbm.at[pl.ds(base + k * _ZR, _ZR)])

    plsc.subcore_barrier()

    @pl.when(wid == 0)
    def _finish():
        pltpu.sync_copy(shared, comb_v)
        tpb = _NSUB // _B
        den = [jnp.full((_E,), 1e-4, jnp.float32),
               jnp.full((_E,), 1e-4, jnp.float32)]
        ts = []
        for b in range(_B):
            sel = []
            for j in range(_K):
                acc = comb_v[b * tpb, j]
                for t in range(1, tpb):
                    acc = jnp.maximum(acc, comb_v[b * tpb + t, j])
                sel.append(acc)
            pltpu.sync_copy(med_hbm.at[pl.ds(b * _L, _E)], rows_v)
            for l in range(_E):
                v = rows_v[l]
                e = jnp.exp(v - jnp.max(v))
                g_v[l] = e / jnp.sum(e)
            tb = []
            for j in range(_K):
                colj = plsc.load_gather(
                    g_v, [iota, jnp.full((_E,), j, jnp.int32)])
                tj = colj * sel[j]
                den[j] = den[j] + tj
                tb.append(tj)
            ts.append(tb)
        for b in range(_B):
            for l in range(_E):
                ob_v[l] = zero
            for j in range(_K):
                oj = ts[b][j] / den[j] * _CAP
                plsc.store_scatter(
                    ob_v, [iota, jnp.full((_E,), j, jnp.int32)], oj)
            pltpu.sync_copy(ob_v, out_hbm.at[pl.ds(b * _L, _E)])


def kernel(x, Wt, bt, gt, bet, Ws, bs, gs, bes, W2, b2):
    mm = _moving_mean(x)
    x2 = x.reshape(_NT, _D)
    mm2 = mm.reshape(_NT, _D)
    med = _dense(x2, mm2, Wt, bt.reshape(1, _D), gt.reshape(1, _D),
                 bet.reshape(1, _D), Ws, bs.reshape(1, _D), gs.reshape(1, _D),
                 bes.reshape(1, _D), W2, b2.reshape(1, _E))
    out2 = _build_route_kernel()(med)
    return out2.reshape(_B, _L, _E)
